# baseline (speedup 1.0000x reference)
import functools

import jax
import jax.numpy as jnp
from jax import lax
from jax.experimental import pallas as pl
from jax.experimental.pallas import tpu as pltpu
from jax.experimental.pallas import tpu_sc as plsc

F32 = jnp.float32


def _dot(a, b):
    return jnp.dot(a.astype(jnp.bfloat16), b.astype(jnp.bfloat16),
                   preferred_element_type=F32)


def _layernorm(h, g, b):
    mu = jnp.mean(h, axis=-1, keepdims=True)
    var = jnp.mean((h - mu) ** 2, axis=-1, keepdims=True)
    return (h - mu) * lax.rsqrt(var + 1e-5) * g + b


def _emb_body(ea_ref, w1_ref, b1_ref, w2_ref, b2_ref, g_ref, be_ref, o_ref):
    h = _dot(ea_ref[...], w1_ref[...]) + b1_ref[...]
    h = h * jax.nn.sigmoid(h)
    h = _dot(h, w2_ref[...]) + b2_ref[...]
    o_ref[...] = _layernorm(h, g_ref[...], be_ref[...])


def _emb_mlp(ea, p, row_off, rows):
    n, d_in = ea.shape
    d = p['w2'].shape[1]
    blk = 2000
    grid = rows // blk
    off = row_off // blk
    return pl.pallas_call(
        _emb_body,
        grid=(grid,),
        in_specs=[
            pl.BlockSpec((blk, d_in), lambda i: (i + off, 0)),
            pl.BlockSpec((d_in, d), lambda i: (0, 0)),
            pl.BlockSpec((1, d), lambda i: (0, 0)),
            pl.BlockSpec((d, d), lambda i: (0, 0)),
            pl.BlockSpec((1, d), lambda i: (0, 0)),
            pl.BlockSpec((1, d), lambda i: (0, 0)),
            pl.BlockSpec((1, d), lambda i: (0, 0)),
        ],
        out_specs=pl.BlockSpec((blk, d), lambda i: (i, 0)),
        out_shape=jax.ShapeDtypeStruct((rows, d), F32),
    )(ea, p['w1'], p['b1'].reshape(1, d), p['w2'], p['b2'].reshape(1, d),
      p['ln_g'].reshape(1, d), p['ln_b'].reshape(1, d))


def _pack_bf16(y):
    a = lax.bitcast_convert_type(y[:, :64].astype(jnp.bfloat16), jnp.uint16)
    b = lax.bitcast_convert_type(y[:, 64:].astype(jnp.bfloat16), jnp.uint16)
    packed = a.astype(jnp.uint32) | (b.astype(jnp.uint32) << 16)
    return lax.bitcast_convert_type(packed, jnp.int32)


def _unpack_bf16(p):
    u = lax.bitcast_convert_type(p, jnp.uint32)
    lo = lax.bitcast_convert_type(u << 16, F32)
    hi = lax.bitcast_convert_type(u & jnp.uint32(0xFFFF0000), F32)
    return jnp.concatenate([lo, hi], axis=1)


def _pair_linear_body(x_ref, wi_ref, wj_ref, yd_ref, ys_ref):
    x = x_ref[...]
    yd_ref[...] = _dot(x, wi_ref[...])
    ys_ref[...] = _dot(x, wj_ref[...])


def _pair_linear(x, wi, wj):
    n, d = x.shape
    blk = 2000
    return pl.pallas_call(
        _pair_linear_body,
        grid=(n // blk,),
        in_specs=[
            pl.BlockSpec((blk, d), lambda i: (i, 0)),
            pl.BlockSpec((d, d), lambda i: (0, 0)),
            pl.BlockSpec((d, d), lambda i: (0, 0)),
        ],
        out_specs=[
            pl.BlockSpec((blk, d), lambda i: (i, 0)),
            pl.BlockSpec((blk, d), lambda i: (i, 0)),
        ],
        out_shape=[
            jax.ShapeDtypeStruct((n, d), F32),
            jax.ShapeDtypeStruct((n, d), F32),
        ],
    )(x, wi, wj)


def _edge_body(gd_ref, gs_ref, ea_ref, w1e_ref, b1_ref, w2_ref, b2_ref,
               g_ref, be_ref, o_ref):
    ea = ea_ref[...]
    h = gd_ref[...] + gs_ref[...] + b1_ref[...]
    h = h + _dot(ea, w1e_ref[...])
    h = h * jax.nn.sigmoid(h)
    h = _dot(h, w2_ref[...]) + b2_ref[...]
    o_ref[...] = _layernorm(h, g_ref[...], be_ref[...]) + ea


def _edge_mlp(gd, gs, ea, p):
    n, d = ea.shape
    w1e = p['w1'][2 * d:, :]
    blk = 2000
    return pl.pallas_call(
        _edge_body,
        grid=(n // blk,),
        in_specs=[
            pl.BlockSpec((blk, d), lambda i: (i, 0)),
            pl.BlockSpec((blk, d), lambda i: (i, 0)),
            pl.BlockSpec((blk, d), lambda i: (i, 0)),
            pl.BlockSpec((d, d), lambda i: (0, 0)),
            pl.BlockSpec((1, d), lambda i: (0, 0)),
            pl.BlockSpec((d, d), lambda i: (0, 0)),
            pl.BlockSpec((1, d), lambda i: (0, 0)),
            pl.BlockSpec((1, d), lambda i: (0, 0)),
            pl.BlockSpec((1, d), lambda i: (0, 0)),
        ],
        out_specs=pl.BlockSpec((blk, d), lambda i: (i, 0)),
        out_shape=jax.ShapeDtypeStruct((n, d), F32),
    )(gd, gs, ea, w1e, p['b1'].reshape(1, d), p['w2'], p['b2'].reshape(1, d),
      p['ln_g'].reshape(1, d), p['ln_b'].reshape(1, d))


def _node_body(x_ref, p0_ref, p1_ref, p2_ref, p3_ref, v1x_ref, v1a_ref,
               c1_ref, v2_ref, c2_ref, g_ref, be_ref, o_ref):
    x = x_ref[...]
    agg = (p0_ref[...] + p1_ref[...]) + (p2_ref[...] + p3_ref[...])
    h = _dot(x, v1x_ref[...]) + c1_ref[...]
    h = h + _dot(agg, v1a_ref[...])
    h = h * jax.nn.sigmoid(h)
    h = _dot(h, v2_ref[...]) + c2_ref[...]
    o_ref[...] = _layernorm(h, g_ref[...], be_ref[...]) + x


def _node_mlp(x, parts_list, p):
    n, d = x.shape
    v1x = p['w1'][:d, :]
    v1a = p['w1'][d:, :]
    blk = 2000
    p0, p1 = parts_list[0][0], parts_list[0][1]
    p2, p3 = parts_list[1][0], parts_list[1][1]
    return pl.pallas_call(
        _node_body,
        grid=(n // blk,),
        in_specs=[
            pl.BlockSpec((blk, d), lambda i: (i, 0)),
            pl.BlockSpec((blk, d), lambda i: (i, 0)),
            pl.BlockSpec((blk, d), lambda i: (i, 0)),
            pl.BlockSpec((blk, d), lambda i: (i, 0)),
            pl.BlockSpec((blk, d), lambda i: (i, 0)),
            pl.BlockSpec((d, d), lambda i: (0, 0)),
            pl.BlockSpec((d, d), lambda i: (0, 0)),
            pl.BlockSpec((1, d), lambda i: (0, 0)),
            pl.BlockSpec((d, d), lambda i: (0, 0)),
            pl.BlockSpec((1, d), lambda i: (0, 0)),
            pl.BlockSpec((1, d), lambda i: (0, 0)),
            pl.BlockSpec((1, d), lambda i: (0, 0)),
        ],
        out_specs=pl.BlockSpec((blk, d), lambda i: (i, 0)),
        out_shape=jax.ShapeDtypeStruct((n, d), F32),
    )(x, p0, p1, p2, p3, v1x, v1a, p['b1'].reshape(1, d), p['w2'],
      p['b2'].reshape(1, d), p['ln_g'].reshape(1, d), p['ln_b'].reshape(1, d))


_NC = 2
_NS = 16
_NW = _NC * _NS
_W = 128


def _sc_gather_spmem(table, idx2d):
    n, d = table.shape
    n_edges = idx2d.shape[1]
    nblk = n_edges // _W
    main = (nblk // _NW) * _NW
    tail_blocks = nblk - main
    rows_per_sub = (n // _NS) // 8 * 8
    tail_start = rows_per_sub * _NS
    tail_rows = n - tail_start
    mesh = plsc.VectorSubcoreMesh(core_axis_name="c", subcore_axis_name="s")

    @functools.partial(
        pl.kernel, mesh=mesh,
        out_type=jax.ShapeDtypeStruct((n_edges, d), table.dtype),
        scratch_types=[
            pltpu.VMEM_SHARED((n, d), table.dtype),
            pltpu.VMEM((_W,), jnp.int32),
            pltpu.VMEM((_W, d), table.dtype),
        ],
    )
    def k(tbl_hbm, di_hbm, out_hbm, tbl_sh, idx_tv, rows_tv):
        cid = lax.axis_index("c")
        sid = lax.axis_index("s")
        r0 = sid * rows_per_sub
        pltpu.sync_copy(tbl_hbm.at[pl.ds(r0, rows_per_sub)],
                        tbl_sh.at[pl.ds(r0, rows_per_sub)])

        @pl.when(sid == 0)
        def _():
            pltpu.sync_copy(tbl_hbm.at[pl.ds(tail_start, tail_rows)],
                            tbl_sh.at[pl.ds(tail_start, tail_rows)])

        plsc.subcore_barrier()

        def body(di_v, o_v):
            pltpu.sync_copy(tbl_sh.at[di_v.at[0]], o_v)

        pltpu.emit_pipeline(
            body,
            grid=(main,),
            in_specs=[pl.BlockSpec((1, _W), lambda i: (0, i))],
            out_specs=[pl.BlockSpec((_W, d), lambda i: (i, 0))],
            core_axis_name=("c", "s"),
            dimension_semantics=(pltpu.PARALLEL,),
        )(di_hbm, out_hbm)

        wid = sid * _NC + cid

        @pl.when(wid < tail_blocks)
        def _():
            base = (main + wid) * _W
            pltpu.sync_copy(di_hbm.at[0].at[pl.ds(base, _W)], idx_tv)
            pltpu.sync_copy(tbl_sh.at[idx_tv], rows_tv)
            pltpu.sync_copy(rows_tv, out_hbm.at[pl.ds(base, _W)])

    return k(table, idx2d)


def _sc_segsum(msgs, idx2d, zeros):
    n_edges, d = msgs.shape
    n = zeros.shape[0]
    nblk = n_edges // _W
    main = (nblk // _NW) * _NW
    tail_blocks = nblk - main
    rows_per_sub = (n // _NS) // 8 * 8
    tail_start = rows_per_sub * _NS
    tail = n - tail_start
    mesh = plsc.VectorSubcoreMesh(core_axis_name="c", subcore_axis_name="s")

    @functools.partial(
        pl.kernel, mesh=mesh,
        out_type=jax.ShapeDtypeStruct((_NC, n, d), F32),
        scratch_types=[
            pltpu.VMEM((_W,), jnp.int32),
            pltpu.VMEM((_W, d), F32),
            pltpu.VMEM_SHARED((n, d), F32),
        ],
    )
    def k(msgs_hbm, idx_hbm, zeros_hbm, out_hbm, idx_v, rows_v, agg_sh):
        cid = lax.axis_index("c")
        sid = lax.axis_index("s")
        wid = sid * _NC + cid
        r0 = sid * rows_per_sub
        pltpu.sync_copy(zeros_hbm.at[pl.ds(r0, rows_per_sub)],
                        agg_sh.at[pl.ds(r0, rows_per_sub)])

        @pl.when(sid == 0)
        def _():
            pltpu.sync_copy(zeros_hbm.at[pl.ds(tail_start, tail)],
                            agg_sh.at[pl.ds(tail_start, tail)])

        plsc.subcore_barrier()

        def body(m_v, di_v):
            pltpu.sync_copy(m_v, agg_sh.at[di_v.at[0]], add=True)

        pltpu.emit_pipeline(
            body,
            grid=(main,),
            in_specs=[
                pl.BlockSpec((_W, d), lambda i: (i, 0)),
                pl.BlockSpec((1, _W), lambda i: (0, i)),
            ],
            out_specs=[],
            core_axis_name=("c", "s"),
            dimension_semantics=(pltpu.PARALLEL,),
        )(msgs_hbm, idx_hbm)

        @pl.when(wid < tail_blocks)
        def _():
            base = (main + wid) * _W
            pltpu.sync_copy(idx_hbm.at[0].at[pl.ds(base, _W)], idx_v)
            pltpu.sync_copy(msgs_hbm.at[pl.ds(base, _W)], rows_v)
            pltpu.sync_copy(rows_v, agg_sh.at[idx_v], add=True)

        plsc.subcore_barrier()
        pltpu.sync_copy(agg_sh.at[pl.ds(r0, rows_per_sub)],
                        out_hbm.at[cid].at[pl.ds(r0, rows_per_sub)])

        @pl.when(sid == 0)
        def _():
            pltpu.sync_copy(agg_sh.at[pl.ds(tail_start, tail)],
                            out_hbm.at[cid].at[pl.ds(tail_start, tail)])

    return k(msgs, idx2d, zeros)


def kernel(x, edge_attr, edge_index, shapes, emb_params, block_params):
    del shapes
    n, d = x.shape
    n_edges = edge_index.shape[1]
    half = n_edges // 2
    src2d = edge_index[0].reshape(1, -1)
    dst2d = edge_index[1].reshape(1, -1)
    srcs = [src2d[:, :half], src2d[:, half:]]
    dsts = [dst2d[:, :half], dst2d[:, half:]]
    zeros = jnp.zeros((n, d), F32)

    eas = [_emb_mlp(edge_attr, emb_params, 0, half),
           _emb_mlp(edge_attr, emb_params, half, half)]
    x_out = x
    for p in block_params:
        w1 = p['edge_mlp']['w1']
        yd, ys = _pair_linear(x_out, w1[:d, :], w1[d:2 * d, :])
        ens, parts = [], []
        for k in range(2):
            gd = _sc_gather_spmem(yd, dsts[k])
            gs = _sc_gather_spmem(ys, srcs[k])
            ens.append(_edge_mlp(gd, gs, eas[k], p['edge_mlp']))
            parts.append(_sc_segsum(ens[k], dsts[k], zeros))
        x_out = _node_mlp(x_out, parts, p['node_mlp'])
        eas = ens
    return (x_out, jnp.concatenate(eas, axis=0))


# speedup vs baseline: 1.0839x; 1.0839x over previous
import functools

import jax
import jax.numpy as jnp
from jax import lax
from jax.experimental import pallas as pl
from jax.experimental.pallas import tpu as pltpu
from jax.experimental.pallas import tpu_sc as plsc

F32 = jnp.float32


def _dot(a, b):
    return jnp.dot(a, b, preferred_element_type=F32)


def _layernorm(h, g, b):
    mu = jnp.mean(h, axis=-1, keepdims=True)
    var = jnp.mean((h - mu) ** 2, axis=-1, keepdims=True)
    return (h - mu) * lax.rsqrt(var + 1e-5) * g + b


def _emb_body(h1_ref, w2_ref, b2_ref, g_ref, be_ref, o_ref):
    h = h1_ref[...]
    h = h * jax.nn.sigmoid(h)
    h = _dot(h, w2_ref[...]) + b2_ref[...]
    o_ref[...] = _layernorm(h, g_ref[...], be_ref[...])


def _emb_mlp(h1, p, row_off, rows):
    d = p['w2'].shape[1]
    blk = 2000
    grid = rows // blk
    off = row_off // blk
    return pl.pallas_call(
        _emb_body,
        grid=(grid,),
        in_specs=[
            pl.BlockSpec((blk, d), lambda i: (i + off, 0)),
            pl.BlockSpec((d, d), lambda i: (0, 0)),
            pl.BlockSpec((1, d), lambda i: (0, 0)),
            pl.BlockSpec((1, d), lambda i: (0, 0)),
            pl.BlockSpec((1, d), lambda i: (0, 0)),
        ],
        out_specs=pl.BlockSpec((blk, d), lambda i: (i, 0)),
        out_shape=jax.ShapeDtypeStruct((rows, d), F32),
    )(h1, p['w2'], p['b2'].reshape(1, d),
      p['ln_g'].reshape(1, d), p['ln_b'].reshape(1, d))


def _pack_bf16(y):
    a = lax.bitcast_convert_type(y[:, :64].astype(jnp.bfloat16), jnp.uint16)
    b = lax.bitcast_convert_type(y[:, 64:].astype(jnp.bfloat16), jnp.uint16)
    packed = a.astype(jnp.uint32) | (b.astype(jnp.uint32) << 16)
    return lax.bitcast_convert_type(packed, jnp.int32)


def _unpack_bf16(p):
    u = lax.bitcast_convert_type(p, jnp.uint32)
    lo = lax.bitcast_convert_type(u << 16, F32)
    hi = lax.bitcast_convert_type(u & jnp.uint32(0xFFFF0000), F32)
    return jnp.concatenate([lo, hi], axis=1)


def _pair_linear_body(x_ref, wi_ref, wj_ref, yd_ref, ys_ref):
    x = x_ref[...]
    yd_ref[...] = _dot(x, wi_ref[...])
    ys_ref[...] = _dot(x, wj_ref[...])


def _pair_linear(x, wi, wj):
    n, d = x.shape
    blk = 2000
    return pl.pallas_call(
        _pair_linear_body,
        grid=(n // blk,),
        in_specs=[
            pl.BlockSpec((blk, d), lambda i: (i, 0)),
            pl.BlockSpec((d, d), lambda i: (0, 0)),
            pl.BlockSpec((d, d), lambda i: (0, 0)),
        ],
        out_specs=[
            pl.BlockSpec((blk, d), lambda i: (i, 0)),
            pl.BlockSpec((blk, d), lambda i: (i, 0)),
        ],
        out_shape=[
            jax.ShapeDtypeStruct((n, d), F32),
            jax.ShapeDtypeStruct((n, d), F32),
        ],
    )(x, wi, wj)


def _edge_body(gd_ref, gs_ref, ea_ref, w1e_ref, b1_ref, w2_ref, b2_ref,
               g_ref, be_ref, o_ref):
    ea = ea_ref[...]
    h = gd_ref[...] + gs_ref[...] + b1_ref[...]
    h = h + _dot(ea, w1e_ref[...])
    h = h * jax.nn.sigmoid(h)
    h = _dot(h, w2_ref[...]) + b2_ref[...]
    o_ref[...] = _layernorm(h, g_ref[...], be_ref[...]) + ea


def _edge_mlp(gd, gs, ea, p, row_off):
    n, d = ea.shape
    w1e = p['w1'][2 * d:, :]
    blk = 4000
    off = row_off // blk
    return pl.pallas_call(
        _edge_body,
        grid=(n // blk,),
        in_specs=[
            pl.BlockSpec((blk, d), lambda i: (i + off, 0)),
            pl.BlockSpec((blk, d), lambda i: (i + off, 0)),
            pl.BlockSpec((blk, d), lambda i: (i, 0)),
            pl.BlockSpec((d, d), lambda i: (0, 0)),
            pl.BlockSpec((1, d), lambda i: (0, 0)),
            pl.BlockSpec((d, d), lambda i: (0, 0)),
            pl.BlockSpec((1, d), lambda i: (0, 0)),
            pl.BlockSpec((1, d), lambda i: (0, 0)),
            pl.BlockSpec((1, d), lambda i: (0, 0)),
        ],
        out_specs=pl.BlockSpec((blk, d), lambda i: (i, 0)),
        out_shape=jax.ShapeDtypeStruct((n, d), F32),
    )(gd, gs, ea, w1e, p['b1'].reshape(1, d), p['w2'], p['b2'].reshape(1, d),
      p['ln_g'].reshape(1, d), p['ln_b'].reshape(1, d))


def _node_body(x_ref, p0_ref, p1_ref, p2_ref, p3_ref, v1x_ref, v1a_ref,
               c1_ref, v2_ref, c2_ref, g_ref, be_ref, o_ref):
    x = x_ref[...]
    agg = (p0_ref[...] + p1_ref[...]) + (p2_ref[...] + p3_ref[...])
    h = _dot(x, v1x_ref[...]) + c1_ref[...]
    h = h + _dot(agg, v1a_ref[...])
    h = h * jax.nn.sigmoid(h)
    h = _dot(h, v2_ref[...]) + c2_ref[...]
    o_ref[...] = _layernorm(h, g_ref[...], be_ref[...]) + x


def _node_mlp(x, parts_list, p):
    n, d = x.shape
    v1x = p['w1'][:d, :]
    v1a = p['w1'][d:, :]
    blk = 2000
    p0, p1 = parts_list[0][0], parts_list[0][1]
    p2, p3 = parts_list[1][0], parts_list[1][1]
    return pl.pallas_call(
        _node_body,
        grid=(n // blk,),
        in_specs=[
            pl.BlockSpec((blk, d), lambda i: (i, 0)),
            pl.BlockSpec((blk, d), lambda i: (i, 0)),
            pl.BlockSpec((blk, d), lambda i: (i, 0)),
            pl.BlockSpec((blk, d), lambda i: (i, 0)),
            pl.BlockSpec((blk, d), lambda i: (i, 0)),
            pl.BlockSpec((d, d), lambda i: (0, 0)),
            pl.BlockSpec((d, d), lambda i: (0, 0)),
            pl.BlockSpec((1, d), lambda i: (0, 0)),
            pl.BlockSpec((d, d), lambda i: (0, 0)),
            pl.BlockSpec((1, d), lambda i: (0, 0)),
            pl.BlockSpec((1, d), lambda i: (0, 0)),
            pl.BlockSpec((1, d), lambda i: (0, 0)),
        ],
        out_specs=pl.BlockSpec((blk, d), lambda i: (i, 0)),
        out_shape=jax.ShapeDtypeStruct((n, d), F32),
    )(x, p0, p1, p2, p3, v1x, v1a, p['b1'].reshape(1, d), p['w2'],
      p['b2'].reshape(1, d), p['ln_g'].reshape(1, d), p['ln_b'].reshape(1, d))


_NC = 2
_NS = 16
_NW = _NC * _NS
_W = 128


def _sc_gather_spmem(table, idx2d):
    n, d = table.shape
    n_edges = idx2d.shape[1]
    nblk = n_edges // _W
    main = (nblk // _NW) * _NW
    tail_blocks = nblk - main
    rows_per_sub = (n // _NS) // 8 * 8
    tail_start = rows_per_sub * _NS
    tail_rows = n - tail_start
    mesh = plsc.VectorSubcoreMesh(core_axis_name="c", subcore_axis_name="s")

    @functools.partial(
        pl.kernel, mesh=mesh,
        out_type=jax.ShapeDtypeStruct((n_edges, d), table.dtype),
        scratch_types=[
            pltpu.VMEM_SHARED((n, d), table.dtype),
            pltpu.VMEM((_W,), jnp.int32),
            pltpu.VMEM((_W, d), table.dtype),
        ],
    )
    def k(tbl_hbm, di_hbm, out_hbm, tbl_sh, idx_tv, rows_tv):
        cid = lax.axis_index("c")
        sid = lax.axis_index("s")
        r0 = sid * rows_per_sub
        pltpu.sync_copy(tbl_hbm.at[pl.ds(r0, rows_per_sub)],
                        tbl_sh.at[pl.ds(r0, rows_per_sub)])

        @pl.when(sid == 0)
        def _():
            pltpu.sync_copy(tbl_hbm.at[pl.ds(tail_start, tail_rows)],
                            tbl_sh.at[pl.ds(tail_start, tail_rows)])

        plsc.subcore_barrier()

        def body(di_v, o_v):
            pltpu.sync_copy(tbl_sh.at[di_v.at[0]], o_v)

        pltpu.emit_pipeline(
            body,
            grid=(main,),
            in_specs=[pl.BlockSpec((1, _W), lambda i: (0, i))],
            out_specs=[pl.BlockSpec((_W, d), lambda i: (i, 0))],
            core_axis_name=("c", "s"),
            dimension_semantics=(pltpu.PARALLEL,),
        )(di_hbm, out_hbm)

        wid = sid * _NC + cid

        @pl.when(wid < tail_blocks)
        def _():
            base = (main + wid) * _W
            pltpu.sync_copy(di_hbm.at[0].at[pl.ds(base, _W)], idx_tv)
            pltpu.sync_copy(tbl_sh.at[idx_tv], rows_tv)
            pltpu.sync_copy(rows_tv, out_hbm.at[pl.ds(base, _W)])

    return k(table, idx2d)


def _sc_segsum(msgs, idx2d, zeros):
    n_edges, d = msgs.shape
    n = zeros.shape[0]
    nblk = n_edges // _W
    main = (nblk // _NW) * _NW
    tail_blocks = nblk - main
    rows_per_sub = (n // _NS) // 8 * 8
    tail_start = rows_per_sub * _NS
    tail = n - tail_start
    mesh = plsc.VectorSubcoreMesh(core_axis_name="c", subcore_axis_name="s")

    @functools.partial(
        pl.kernel, mesh=mesh,
        out_type=jax.ShapeDtypeStruct((_NC, n, d), F32),
        scratch_types=[
            pltpu.VMEM((_W,), jnp.int32),
            pltpu.VMEM((_W, d), F32),
            pltpu.VMEM_SHARED((n, d), F32),
        ],
    )
    def k(msgs_hbm, idx_hbm, zeros_hbm, out_hbm, idx_v, rows_v, agg_sh):
        cid = lax.axis_index("c")
        sid = lax.axis_index("s")
        wid = sid * _NC + cid
        r0 = sid * rows_per_sub
        pltpu.sync_copy(zeros_hbm.at[pl.ds(r0, rows_per_sub)],
                        agg_sh.at[pl.ds(r0, rows_per_sub)])

        @pl.when(sid == 0)
        def _():
            pltpu.sync_copy(zeros_hbm.at[pl.ds(tail_start, tail)],
                            agg_sh.at[pl.ds(tail_start, tail)])

        plsc.subcore_barrier()

        def body(m_v, di_v):
            pltpu.sync_copy(m_v, agg_sh.at[di_v.at[0]], add=True)

        pltpu.emit_pipeline(
            body,
            grid=(main,),
            in_specs=[
                pl.BlockSpec((_W, d), lambda i: (i, 0)),
                pl.BlockSpec((1, _W), lambda i: (0, i)),
            ],
            out_specs=[],
            core_axis_name=("c", "s"),
            dimension_semantics=(pltpu.PARALLEL,),
        )(msgs_hbm, idx_hbm)

        @pl.when(wid < tail_blocks)
        def _():
            base = (main + wid) * _W
            pltpu.sync_copy(idx_hbm.at[0].at[pl.ds(base, _W)], idx_v)
            pltpu.sync_copy(msgs_hbm.at[pl.ds(base, _W)], rows_v)
            pltpu.sync_copy(rows_v, agg_sh.at[idx_v], add=True)

        plsc.subcore_barrier()
        pltpu.sync_copy(agg_sh.at[pl.ds(r0, rows_per_sub)],
                        out_hbm.at[cid].at[pl.ds(r0, rows_per_sub)])

        @pl.when(sid == 0)
        def _():
            pltpu.sync_copy(agg_sh.at[pl.ds(tail_start, tail)],
                            out_hbm.at[cid].at[pl.ds(tail_start, tail)])

    return k(msgs, idx2d, zeros)


def kernel(x, edge_attr, edge_index, shapes, emb_params, block_params):
    del shapes
    n, d = x.shape
    n_edges = edge_index.shape[1]
    half = n_edges // 2
    src2d = edge_index[0].reshape(1, -1)
    dst2d = edge_index[1].reshape(1, -1)
    srcs = [src2d[:, :half], src2d[:, half:]]
    dsts = [dst2d[:, :half], dst2d[:, half:]]
    zeros = jnp.zeros((n, d), F32)

    h1 = edge_attr @ emb_params['w1'] + emb_params['b1']
    eas = [_emb_mlp(h1, emb_params, 0, half),
           _emb_mlp(h1, emb_params, half, half)]
    x_out = x
    for p in block_params:
        w1 = p['edge_mlp']['w1']
        yd, ys = _pair_linear(x_out, w1[:d, :], w1[d:2 * d, :])
        gd = _sc_gather_spmem(yd, dst2d)
        gs = _sc_gather_spmem(ys, src2d)
        ens, parts = [], []
        for k in range(2):
            ens.append(_edge_mlp(gd, gs, eas[k], p['edge_mlp'], k * half))
            parts.append(_sc_segsum(ens[k], dsts[k], zeros))
        x_out = _node_mlp(x_out, parts, p['node_mlp'])
        eas = ens
    return (x_out, jnp.concatenate(eas, axis=0))


# speedup vs baseline: 1.1429x; 1.0544x over previous
import functools

import jax
import jax.numpy as jnp
from jax import lax
from jax.experimental import pallas as pl
from jax.experimental.pallas import tpu as pltpu
from jax.experimental.pallas import tpu_sc as plsc

F32 = jnp.float32


def _dot(a, b):
    return jnp.dot(a, b, preferred_element_type=F32)


def _layernorm(h, g, b):
    mu = jnp.mean(h, axis=-1, keepdims=True)
    var = jnp.mean((h - mu) ** 2, axis=-1, keepdims=True)
    return (h - mu) * lax.rsqrt(var + 1e-5) * g + b


def _emb_body(h1_ref, w2_ref, b2_ref, g_ref, be_ref, o_ref):
    h = h1_ref[...]
    h = h * jax.nn.sigmoid(h)
    h = _dot(h, w2_ref[...]) + b2_ref[...]
    o_ref[...] = _layernorm(h, g_ref[...], be_ref[...])


def _emb_mlp(h1, p, row_off, rows):
    d = p['w2'].shape[1]
    blk = 8000
    grid = rows // blk
    off = row_off // blk
    return pl.pallas_call(
        _emb_body,
        grid=(grid,),
        in_specs=[
            pl.BlockSpec((blk, d), lambda i: (i + off, 0)),
            pl.BlockSpec((d, d), lambda i: (0, 0)),
            pl.BlockSpec((1, d), lambda i: (0, 0)),
            pl.BlockSpec((1, d), lambda i: (0, 0)),
            pl.BlockSpec((1, d), lambda i: (0, 0)),
        ],
        out_specs=pl.BlockSpec((blk, d), lambda i: (i, 0)),
        out_shape=jax.ShapeDtypeStruct((rows, d), F32),
    )(h1, p['w2'], p['b2'].reshape(1, d),
      p['ln_g'].reshape(1, d), p['ln_b'].reshape(1, d))


def _pack_bf16(y):
    a = lax.bitcast_convert_type(y[:, :64].astype(jnp.bfloat16), jnp.uint16)
    b = lax.bitcast_convert_type(y[:, 64:].astype(jnp.bfloat16), jnp.uint16)
    packed = a.astype(jnp.uint32) | (b.astype(jnp.uint32) << 16)
    return lax.bitcast_convert_type(packed, jnp.int32)


def _unpack_bf16(p):
    u = lax.bitcast_convert_type(p, jnp.uint32)
    lo = lax.bitcast_convert_type(u << 16, F32)
    hi = lax.bitcast_convert_type(u & jnp.uint32(0xFFFF0000), F32)
    return jnp.concatenate([lo, hi], axis=1)


def _pair_linear_body(x_ref, wi_ref, wj_ref, yd_ref, ys_ref):
    x = x_ref[...]
    yd_ref[...] = _dot(x, wi_ref[...])
    ys_ref[...] = _dot(x, wj_ref[...])


def _pair_linear(x, wi, wj):
    n, d = x.shape
    blk = 2000
    return pl.pallas_call(
        _pair_linear_body,
        grid=(n // blk,),
        in_specs=[
            pl.BlockSpec((blk, d), lambda i: (i, 0)),
            pl.BlockSpec((d, d), lambda i: (0, 0)),
            pl.BlockSpec((d, d), lambda i: (0, 0)),
        ],
        out_specs=[
            pl.BlockSpec((blk, d), lambda i: (i, 0)),
            pl.BlockSpec((blk, d), lambda i: (i, 0)),
        ],
        out_shape=[
            jax.ShapeDtypeStruct((n, d), F32),
            jax.ShapeDtypeStruct((n, d), F32),
        ],
    )(x, wi, wj)


def _edge_body(gd_ref, gs_ref, ea_ref, w1e_ref, b1_ref, w2_ref, b2_ref,
               g_ref, be_ref, o_ref):
    ea = ea_ref[...]
    h = gd_ref[...] + gs_ref[...] + b1_ref[...]
    h = h + _dot(ea, w1e_ref[...])
    h = h * jax.nn.sigmoid(h)
    h = _dot(h, w2_ref[...]) + b2_ref[...]
    o_ref[...] = _layernorm(h, g_ref[...], be_ref[...]) + ea


def _edge_mlp(gd, gs, ea, p, rows, g_off, ea_off):
    d = ea.shape[1]
    w1e = p['w1'][2 * d:, :]
    blk = 8000
    goff = g_off // blk
    eoff = ea_off // blk
    return pl.pallas_call(
        _edge_body,
        grid=(rows // blk,),
        in_specs=[
            pl.BlockSpec((blk, d), lambda i: (i + goff, 0)),
            pl.BlockSpec((blk, d), lambda i: (i + goff, 0)),
            pl.BlockSpec((blk, d), lambda i: (i + eoff, 0)),
            pl.BlockSpec((d, d), lambda i: (0, 0)),
            pl.BlockSpec((1, d), lambda i: (0, 0)),
            pl.BlockSpec((d, d), lambda i: (0, 0)),
            pl.BlockSpec((1, d), lambda i: (0, 0)),
            pl.BlockSpec((1, d), lambda i: (0, 0)),
            pl.BlockSpec((1, d), lambda i: (0, 0)),
        ],
        out_specs=pl.BlockSpec((blk, d), lambda i: (i, 0)),
        out_shape=jax.ShapeDtypeStruct((rows, d), F32),
    )(gd, gs, ea, w1e, p['b1'].reshape(1, d), p['w2'], p['b2'].reshape(1, d),
      p['ln_g'].reshape(1, d), p['ln_b'].reshape(1, d))


def _node_body(x_ref, p0_ref, p1_ref, p2_ref, p3_ref, v1x_ref, v1a_ref,
               c1_ref, v2_ref, c2_ref, g_ref, be_ref, o_ref):
    x = x_ref[...]
    agg = (p0_ref[...] + p1_ref[...]) + (p2_ref[...] + p3_ref[...])
    h = _dot(x, v1x_ref[...]) + c1_ref[...]
    h = h + _dot(agg, v1a_ref[...])
    h = h * jax.nn.sigmoid(h)
    h = _dot(h, v2_ref[...]) + c2_ref[...]
    o_ref[...] = _layernorm(h, g_ref[...], be_ref[...]) + x


def _node_mlp(x, parts_list, p):
    n, d = x.shape
    v1x = p['w1'][:d, :]
    v1a = p['w1'][d:, :]
    blk = 2000
    p0, p1 = parts_list[0][0], parts_list[0][1]
    p2, p3 = parts_list[1][0], parts_list[1][1]
    return pl.pallas_call(
        _node_body,
        grid=(n // blk,),
        in_specs=[
            pl.BlockSpec((blk, d), lambda i: (i, 0)),
            pl.BlockSpec((blk, d), lambda i: (i, 0)),
            pl.BlockSpec((blk, d), lambda i: (i, 0)),
            pl.BlockSpec((blk, d), lambda i: (i, 0)),
            pl.BlockSpec((blk, d), lambda i: (i, 0)),
            pl.BlockSpec((d, d), lambda i: (0, 0)),
            pl.BlockSpec((d, d), lambda i: (0, 0)),
            pl.BlockSpec((1, d), lambda i: (0, 0)),
            pl.BlockSpec((d, d), lambda i: (0, 0)),
            pl.BlockSpec((1, d), lambda i: (0, 0)),
            pl.BlockSpec((1, d), lambda i: (0, 0)),
            pl.BlockSpec((1, d), lambda i: (0, 0)),
        ],
        out_specs=pl.BlockSpec((blk, d), lambda i: (i, 0)),
        out_shape=jax.ShapeDtypeStruct((n, d), F32),
    )(x, p0, p1, p2, p3, v1x, v1a, p['b1'].reshape(1, d), p['w2'],
      p['b2'].reshape(1, d), p['ln_g'].reshape(1, d), p['ln_b'].reshape(1, d))


_NC = 2
_NS = 16
_NW = _NC * _NS
_W = 128


def _sc_gather_spmem(table, idx2d):
    n, d = table.shape
    n_edges = idx2d.shape[1]
    nblk = n_edges // _W
    main = (nblk // _NW) * _NW
    tail_blocks = nblk - main
    rows_per_sub = (n // _NS) // 8 * 8
    tail_start = rows_per_sub * _NS
    tail_rows = n - tail_start
    mesh = plsc.VectorSubcoreMesh(core_axis_name="c", subcore_axis_name="s")

    @functools.partial(
        pl.kernel, mesh=mesh,
        out_type=jax.ShapeDtypeStruct((n_edges, d), table.dtype),
        scratch_types=[
            pltpu.VMEM_SHARED((n, d), table.dtype),
            pltpu.VMEM((_W,), jnp.int32),
            pltpu.VMEM((_W, d), table.dtype),
        ],
    )
    def k(tbl_hbm, di_hbm, out_hbm, tbl_sh, idx_tv, rows_tv):
        cid = lax.axis_index("c")
        sid = lax.axis_index("s")
        r0 = sid * rows_per_sub
        pltpu.sync_copy(tbl_hbm.at[pl.ds(r0, rows_per_sub)],
                        tbl_sh.at[pl.ds(r0, rows_per_sub)])

        @pl.when(sid == 0)
        def _():
            pltpu.sync_copy(tbl_hbm.at[pl.ds(tail_start, tail_rows)],
                            tbl_sh.at[pl.ds(tail_start, tail_rows)])

        plsc.subcore_barrier()

        def body(di_v, o_v):
            pltpu.sync_copy(tbl_sh.at[di_v.at[0]], o_v)

        pltpu.emit_pipeline(
            body,
            grid=(main,),
            in_specs=[pl.BlockSpec((1, _W), lambda i: (0, i))],
            out_specs=[pl.BlockSpec((_W, d), lambda i: (i, 0))],
            core_axis_name=("c", "s"),
            dimension_semantics=(pltpu.PARALLEL,),
        )(di_hbm, out_hbm)

        wid = sid * _NC + cid

        @pl.when(wid < tail_blocks)
        def _():
            base = (main + wid) * _W
            pltpu.sync_copy(di_hbm.at[0].at[pl.ds(base, _W)], idx_tv)
            pltpu.sync_copy(tbl_sh.at[idx_tv], rows_tv)
            pltpu.sync_copy(rows_tv, out_hbm.at[pl.ds(base, _W)])

    return k(table, idx2d)


def _sc_segsum(msgs, idx2d, zeros):
    n_edges, d = msgs.shape
    n = zeros.shape[0]
    nblk = n_edges // _W
    main = (nblk // _NW) * _NW
    tail_blocks = nblk - main
    rows_per_sub = (n // _NS) // 8 * 8
    tail_start = rows_per_sub * _NS
    tail = n - tail_start
    mesh = plsc.VectorSubcoreMesh(core_axis_name="c", subcore_axis_name="s")

    @functools.partial(
        pl.kernel, mesh=mesh,
        out_type=jax.ShapeDtypeStruct((_NC, n, d), F32),
        scratch_types=[
            pltpu.VMEM((_W,), jnp.int32),
            pltpu.VMEM((_W, d), F32),
            pltpu.VMEM_SHARED((n, d), F32),
        ],
    )
    def k(msgs_hbm, idx_hbm, zeros_hbm, out_hbm, idx_v, rows_v, agg_sh):
        cid = lax.axis_index("c")
        sid = lax.axis_index("s")
        wid = sid * _NC + cid
        r0 = sid * rows_per_sub
        pltpu.sync_copy(zeros_hbm.at[pl.ds(r0, rows_per_sub)],
                        agg_sh.at[pl.ds(r0, rows_per_sub)])

        @pl.when(sid == 0)
        def _():
            pltpu.sync_copy(zeros_hbm.at[pl.ds(tail_start, tail)],
                            agg_sh.at[pl.ds(tail_start, tail)])

        plsc.subcore_barrier()

        def body(m_v, di_v):
            pltpu.sync_copy(m_v, agg_sh.at[di_v.at[0]], add=True)

        pltpu.emit_pipeline(
            body,
            grid=(main,),
            in_specs=[
                pl.BlockSpec((_W, d), lambda i: (i, 0)),
                pl.BlockSpec((1, _W), lambda i: (0, i)),
            ],
            out_specs=[],
            core_axis_name=("c", "s"),
            dimension_semantics=(pltpu.PARALLEL,),
        )(msgs_hbm, idx_hbm)

        @pl.when(wid < tail_blocks)
        def _():
            base = (main + wid) * _W
            pltpu.sync_copy(idx_hbm.at[0].at[pl.ds(base, _W)], idx_v)
            pltpu.sync_copy(msgs_hbm.at[pl.ds(base, _W)], rows_v)
            pltpu.sync_copy(rows_v, agg_sh.at[idx_v], add=True)

        plsc.subcore_barrier()
        pltpu.sync_copy(agg_sh.at[pl.ds(r0, rows_per_sub)],
                        out_hbm.at[cid].at[pl.ds(r0, rows_per_sub)])

        @pl.when(sid == 0)
        def _():
            pltpu.sync_copy(agg_sh.at[pl.ds(tail_start, tail)],
                            out_hbm.at[cid].at[pl.ds(tail_start, tail)])

    return k(msgs, idx2d, zeros)


def kernel(x, edge_attr, edge_index, shapes, emb_params, block_params):
    del shapes
    n, d = x.shape
    n_edges = edge_index.shape[1]
    half = n_edges // 2
    src2d = edge_index[0].reshape(1, -1)
    dst2d = edge_index[1].reshape(1, -1)
    srcs = [src2d[:, :half], src2d[:, half:]]
    dsts = [dst2d[:, :half], dst2d[:, half:]]
    zeros = jnp.zeros((n, d), F32)

    h1 = edge_attr @ emb_params['w1'] + emb_params['b1']
    ea_full = _emb_mlp(h1, emb_params, 0, n_edges)
    eas, ea_offs = [ea_full, ea_full], [0, half]
    x_out = x
    for li, p in enumerate(block_params):
        w1 = p['edge_mlp']['w1']
        yd, ys = _pair_linear(x_out, w1[:d, :], w1[d:2 * d, :])
        if li == 0:
            gd = _sc_gather_spmem(yd, dst2d)
            gs = _sc_gather_spmem(ys, src2d)
            gpair = [(gd, gs, half), (gd, gs, half)]
            g_offs = [0, half]
        else:
            gpair = [( _sc_gather_spmem(yd, dsts[k]),
                       _sc_gather_spmem(ys, srcs[k]), half) for k in range(2)]
            g_offs = [0, 0]
        ens, parts = [], []
        for k in range(2):
            gdk, gsk, rows = gpair[k]
            ens.append(_edge_mlp(gdk, gsk, eas[k], p['edge_mlp'], rows,
                                 g_offs[k], ea_offs[k]))
            parts.append(_sc_segsum(ens[k], dsts[k], zeros))
        x_out = _node_mlp(x_out, parts, p['node_mlp'])
        eas, ea_offs = ens, [0, 0]
    return (x_out, jnp.concatenate(eas, axis=0))


# speedup vs baseline: 1.1433x; 1.0004x over previous
import functools

import jax
import jax.numpy as jnp
from jax import lax
from jax.experimental import pallas as pl
from jax.experimental.pallas import tpu as pltpu
from jax.experimental.pallas import tpu_sc as plsc

F32 = jnp.float32


def _dot(a, b):
    return jnp.dot(a, b, preferred_element_type=F32)


def _layernorm(h, g, b):
    mu = jnp.mean(h, axis=-1, keepdims=True)
    var = jnp.mean((h - mu) ** 2, axis=-1, keepdims=True)
    return (h - mu) * lax.rsqrt(var + 1e-5) * g + b


def _emb_body(h1_ref, w2_ref, b2_ref, g_ref, be_ref, o_ref):
    h = h1_ref[...]
    h = h * jax.nn.sigmoid(h)
    h = _dot(h, w2_ref[...]) + b2_ref[...]
    o_ref[...] = _layernorm(h, g_ref[...], be_ref[...])


def _emb_mlp(h1, p, row_off, rows):
    d = p['w2'].shape[1]
    blk = 8000
    grid = rows // blk
    off = row_off // blk
    return pl.pallas_call(
        _emb_body,
        grid=(grid,),
        in_specs=[
            pl.BlockSpec((blk, d), lambda i: (i + off, 0)),
            pl.BlockSpec((d, d), lambda i: (0, 0)),
            pl.BlockSpec((1, d), lambda i: (0, 0)),
            pl.BlockSpec((1, d), lambda i: (0, 0)),
            pl.BlockSpec((1, d), lambda i: (0, 0)),
        ],
        out_specs=pl.BlockSpec((blk, d), lambda i: (i, 0)),
        out_shape=jax.ShapeDtypeStruct((rows, d), F32),
    )(h1, p['w2'], p['b2'].reshape(1, d),
      p['ln_g'].reshape(1, d), p['ln_b'].reshape(1, d))


def _pack_bf16(y):
    a = lax.bitcast_convert_type(y[:, :64].astype(jnp.bfloat16), jnp.uint16)
    b = lax.bitcast_convert_type(y[:, 64:].astype(jnp.bfloat16), jnp.uint16)
    packed = a.astype(jnp.uint32) | (b.astype(jnp.uint32) << 16)
    return lax.bitcast_convert_type(packed, jnp.int32)


def _unpack_bf16(p):
    u = lax.bitcast_convert_type(p, jnp.uint32)
    lo = lax.bitcast_convert_type(u << 16, F32)
    hi = lax.bitcast_convert_type(u & jnp.uint32(0xFFFF0000), F32)
    return jnp.concatenate([lo, hi], axis=1)


def _pair_linear_body(x_ref, wi_ref, wj_ref, yd_ref, ys_ref):
    x = x_ref[...]
    yd_ref[...] = _dot(x, wi_ref[...])
    ys_ref[...] = _dot(x, wj_ref[...])


def _pair_linear(x, wi, wj):
    n, d = x.shape
    blk = 2000
    return pl.pallas_call(
        _pair_linear_body,
        grid=(n // blk,),
        in_specs=[
            pl.BlockSpec((blk, d), lambda i: (i, 0)),
            pl.BlockSpec((d, d), lambda i: (0, 0)),
            pl.BlockSpec((d, d), lambda i: (0, 0)),
        ],
        out_specs=[
            pl.BlockSpec((blk, d), lambda i: (i, 0)),
            pl.BlockSpec((blk, d), lambda i: (i, 0)),
        ],
        out_shape=[
            jax.ShapeDtypeStruct((n, d), F32),
            jax.ShapeDtypeStruct((n, d), F32),
        ],
    )(x, wi, wj)


def _edge_body(gd_ref, gs_ref, ea_ref, w1e_ref, b1_ref, w2_ref, b2_ref,
               g_ref, be_ref, o_ref):
    ea = ea_ref[...]
    h = gd_ref[...] + gs_ref[...] + b1_ref[...]
    h = h + _dot(ea, w1e_ref[...])
    h = h * jax.nn.sigmoid(h)
    h = _dot(h, w2_ref[...]) + b2_ref[...]
    o_ref[...] = _layernorm(h, g_ref[...], be_ref[...]) + ea


def _edge_mlp(gd, gs, ea, p, rows, g_off, ea_off):
    d = ea.shape[1]
    w1e = p['w1'][2 * d:, :]
    blk = 8000
    goff = g_off // blk
    eoff = ea_off // blk
    return pl.pallas_call(
        _edge_body,
        grid=(rows // blk,),
        in_specs=[
            pl.BlockSpec((blk, d), lambda i: (i + goff, 0)),
            pl.BlockSpec((blk, d), lambda i: (i + goff, 0)),
            pl.BlockSpec((blk, d), lambda i: (i + eoff, 0)),
            pl.BlockSpec((d, d), lambda i: (0, 0)),
            pl.BlockSpec((1, d), lambda i: (0, 0)),
            pl.BlockSpec((d, d), lambda i: (0, 0)),
            pl.BlockSpec((1, d), lambda i: (0, 0)),
            pl.BlockSpec((1, d), lambda i: (0, 0)),
            pl.BlockSpec((1, d), lambda i: (0, 0)),
        ],
        out_specs=pl.BlockSpec((blk, d), lambda i: (i, 0)),
        out_shape=jax.ShapeDtypeStruct((rows, d), F32),
    )(gd, gs, ea, w1e, p['b1'].reshape(1, d), p['w2'], p['b2'].reshape(1, d),
      p['ln_g'].reshape(1, d), p['ln_b'].reshape(1, d))


def _node_body(x_ref, p0_ref, p1_ref, p2_ref, p3_ref, v1x_ref, v1a_ref,
               c1_ref, v2_ref, c2_ref, g_ref, be_ref, o_ref):
    x = x_ref[...]
    agg = (p0_ref[...] + p1_ref[...]) + (p2_ref[...] + p3_ref[...])
    h = _dot(x, v1x_ref[...]) + c1_ref[...]
    h = h + _dot(agg, v1a_ref[...])
    h = h * jax.nn.sigmoid(h)
    h = _dot(h, v2_ref[...]) + c2_ref[...]
    o_ref[...] = _layernorm(h, g_ref[...], be_ref[...]) + x


def _node_mlp(x, parts_list, p):
    n, d = x.shape
    v1x = p['w1'][:d, :]
    v1a = p['w1'][d:, :]
    blk = 2000
    p0, p1 = parts_list[0][0], parts_list[0][1]
    p2, p3 = parts_list[1][0], parts_list[1][1]
    return pl.pallas_call(
        _node_body,
        grid=(n // blk,),
        in_specs=[
            pl.BlockSpec((blk, d), lambda i: (i, 0)),
            pl.BlockSpec((blk, d), lambda i: (i, 0)),
            pl.BlockSpec((blk, d), lambda i: (i, 0)),
            pl.BlockSpec((blk, d), lambda i: (i, 0)),
            pl.BlockSpec((blk, d), lambda i: (i, 0)),
            pl.BlockSpec((d, d), lambda i: (0, 0)),
            pl.BlockSpec((d, d), lambda i: (0, 0)),
            pl.BlockSpec((1, d), lambda i: (0, 0)),
            pl.BlockSpec((d, d), lambda i: (0, 0)),
            pl.BlockSpec((1, d), lambda i: (0, 0)),
            pl.BlockSpec((1, d), lambda i: (0, 0)),
            pl.BlockSpec((1, d), lambda i: (0, 0)),
        ],
        out_specs=pl.BlockSpec((blk, d), lambda i: (i, 0)),
        out_shape=jax.ShapeDtypeStruct((n, d), F32),
    )(x, p0, p1, p2, p3, v1x, v1a, p['b1'].reshape(1, d), p['w2'],
      p['b2'].reshape(1, d), p['ln_g'].reshape(1, d), p['ln_b'].reshape(1, d))


_NC = 2
_NS = 16
_NW = _NC * _NS
_W = 128


def _sc_gather_spmem(table, idx2d):
    n, d = table.shape
    n_edges = idx2d.shape[1]
    nblk = n_edges // _W
    main = (nblk // _NW) * _NW
    tail_blocks = nblk - main
    rows_per_sub = (n // _NS) // 8 * 8
    tail_start = rows_per_sub * _NS
    tail_rows = n - tail_start
    mesh = plsc.VectorSubcoreMesh(core_axis_name="c", subcore_axis_name="s")

    @functools.partial(
        pl.kernel, mesh=mesh,
        out_type=jax.ShapeDtypeStruct((n_edges, d), table.dtype),
        scratch_types=[
            pltpu.VMEM_SHARED((n, d), table.dtype),
            pltpu.VMEM((_W,), jnp.int32),
            pltpu.VMEM((_W, d), table.dtype),
        ],
    )
    def k(tbl_hbm, di_hbm, out_hbm, tbl_sh, idx_tv, rows_tv):
        cid = lax.axis_index("c")
        sid = lax.axis_index("s")
        r0 = sid * rows_per_sub
        pltpu.sync_copy(tbl_hbm.at[pl.ds(r0, rows_per_sub)],
                        tbl_sh.at[pl.ds(r0, rows_per_sub)])

        @pl.when(sid == 0)
        def _():
            pltpu.sync_copy(tbl_hbm.at[pl.ds(tail_start, tail_rows)],
                            tbl_sh.at[pl.ds(tail_start, tail_rows)])

        plsc.subcore_barrier()

        def body(di_v, o_v):
            pltpu.sync_copy(tbl_sh.at[di_v.at[0]], o_v)

        pltpu.emit_pipeline(
            body,
            grid=(main,),
            in_specs=[pl.BlockSpec((1, _W), lambda i: (0, i))],
            out_specs=[pl.BlockSpec((_W, d), lambda i: (i, 0))],
            core_axis_name=("c", "s"),
            dimension_semantics=(pltpu.PARALLEL,),
        )(di_hbm, out_hbm)

        wid = sid * _NC + cid

        @pl.when(wid < tail_blocks)
        def _():
            base = (main + wid) * _W
            pltpu.sync_copy(di_hbm.at[0].at[pl.ds(base, _W)], idx_tv)
            pltpu.sync_copy(tbl_sh.at[idx_tv], rows_tv)
            pltpu.sync_copy(rows_tv, out_hbm.at[pl.ds(base, _W)])

    return k(table, idx2d)


def _sc_segsum(msgs, idx2d, zeros):
    n_edges, d = msgs.shape
    n = zeros.shape[0]
    nblk = n_edges // _W
    main = (nblk // _NW) * _NW
    tail_blocks = nblk - main
    rows_per_sub = (n // _NS) // 8 * 8
    tail_start = rows_per_sub * _NS
    tail = n - tail_start
    mesh = plsc.VectorSubcoreMesh(core_axis_name="c", subcore_axis_name="s")

    @functools.partial(
        pl.kernel, mesh=mesh,
        out_type=jax.ShapeDtypeStruct((_NC, n, d), F32),
        scratch_types=[
            pltpu.VMEM((_W,), jnp.int32),
            pltpu.VMEM((_W, d), F32),
            pltpu.VMEM_SHARED((n, d), F32),
        ],
    )
    def k(msgs_hbm, idx_hbm, zeros_hbm, out_hbm, idx_v, rows_v, agg_sh):
        cid = lax.axis_index("c")
        sid = lax.axis_index("s")
        wid = sid * _NC + cid
        r0 = sid * rows_per_sub
        pltpu.sync_copy(zeros_hbm.at[pl.ds(r0, rows_per_sub)],
                        agg_sh.at[pl.ds(r0, rows_per_sub)])

        @pl.when(sid == 0)
        def _():
            pltpu.sync_copy(zeros_hbm.at[pl.ds(tail_start, tail)],
                            agg_sh.at[pl.ds(tail_start, tail)])

        plsc.subcore_barrier()

        def body(m_v, di_v):
            pltpu.sync_copy(m_v, agg_sh.at[di_v.at[0]], add=True)

        pltpu.emit_pipeline(
            body,
            grid=(main,),
            in_specs=[
                pl.BlockSpec((_W, d), lambda i: (i, 0)),
                pl.BlockSpec((1, _W), lambda i: (0, i)),
            ],
            out_specs=[],
            core_axis_name=("c", "s"),
            dimension_semantics=(pltpu.PARALLEL,),
        )(msgs_hbm, idx_hbm)

        @pl.when(wid < tail_blocks)
        def _():
            base = (main + wid) * _W
            pltpu.sync_copy(idx_hbm.at[0].at[pl.ds(base, _W)], idx_v)
            pltpu.sync_copy(msgs_hbm.at[pl.ds(base, _W)], rows_v)
            pltpu.sync_copy(rows_v, agg_sh.at[idx_v], add=True)

        plsc.subcore_barrier()
        pltpu.sync_copy(agg_sh.at[pl.ds(r0, rows_per_sub)],
                        out_hbm.at[cid].at[pl.ds(r0, rows_per_sub)])

        @pl.when(sid == 0)
        def _():
            pltpu.sync_copy(agg_sh.at[pl.ds(tail_start, tail)],
                            out_hbm.at[cid].at[pl.ds(tail_start, tail)])

    return k(msgs, idx2d, zeros)


def kernel(x, edge_attr, edge_index, shapes, emb_params, block_params):
    del shapes
    n, d = x.shape
    n_edges = edge_index.shape[1]
    half = n_edges // 2
    src2d = edge_index[0].reshape(1, -1)
    dst2d = edge_index[1].reshape(1, -1)
    srcs = [src2d[:, :half], src2d[:, half:]]
    dsts = [dst2d[:, :half], dst2d[:, half:]]
    zeros = jnp.zeros((n, d), F32)

    h1 = jnp.dot(edge_attr.astype(jnp.bfloat16),
                 emb_params['w1'].astype(jnp.bfloat16),
                 preferred_element_type=F32) + emb_params['b1']
    ea_full = _emb_mlp(h1, emb_params, 0, n_edges)
    eas, ea_offs = [ea_full, ea_full], [0, half]
    x_out = x
    for li, p in enumerate(block_params):
        w1 = p['edge_mlp']['w1']
        yd, ys = _pair_linear(x_out, w1[:d, :], w1[d:2 * d, :])
        if li == 0:
            gd = _sc_gather_spmem(yd, dst2d)
            gs = _sc_gather_spmem(ys, src2d)
            gpair = [(gd, gs, half), (gd, gs, half)]
            g_offs = [0, half]
        else:
            gpair = [( _sc_gather_spmem(yd, dsts[k]),
                       _sc_gather_spmem(ys, srcs[k]), half) for k in range(2)]
            g_offs = [0, 0]
        ens, parts = [], []
        for k in range(2):
            gdk, gsk, rows = gpair[k]
            ens.append(_edge_mlp(gdk, gsk, eas[k], p['edge_mlp'], rows,
                                 g_offs[k], ea_offs[k]))
            parts.append(_sc_segsum(ens[k], dsts[k], zeros))
        x_out = _node_mlp(x_out, parts, p['node_mlp'])
        eas, ea_offs = ens, [0, 0]
    return (x_out, jnp.concatenate(eas, axis=0))


# speedup vs baseline: 1.1470x; 1.0032x over previous
import functools

import jax
import jax.numpy as jnp
from jax import lax
from jax.experimental import pallas as pl
from jax.experimental.pallas import tpu as pltpu
from jax.experimental.pallas import tpu_sc as plsc

F32 = jnp.float32


def _dot(a, b):
    return jnp.dot(a, b, preferred_element_type=F32)


def _layernorm(h, g, b):
    mu = jnp.mean(h, axis=-1, keepdims=True)
    var = jnp.mean((h - mu) ** 2, axis=-1, keepdims=True)
    return (h - mu) * lax.rsqrt(var + 1e-5) * g + b


def _emb_body(h1_ref, w2_ref, b2_ref, g_ref, be_ref, o_ref):
    h = h1_ref[...]
    h = h * jax.nn.sigmoid(h)
    h = _dot(h, w2_ref[...]) + b2_ref[...]
    o_ref[...] = _layernorm(h, g_ref[...], be_ref[...])


def _emb_mlp(h1, p, row_off, rows):
    d = p['w2'].shape[1]
    blk = 16000
    grid = rows // blk
    off = row_off // blk
    return pl.pallas_call(
        _emb_body,
        grid=(grid,),
        in_specs=[
            pl.BlockSpec((blk, d), lambda i: (i + off, 0)),
            pl.BlockSpec((d, d), lambda i: (0, 0)),
            pl.BlockSpec((1, d), lambda i: (0, 0)),
            pl.BlockSpec((1, d), lambda i: (0, 0)),
            pl.BlockSpec((1, d), lambda i: (0, 0)),
        ],
        out_specs=pl.BlockSpec((blk, d), lambda i: (i, 0)),
        out_shape=jax.ShapeDtypeStruct((rows, d), F32),
    )(h1, p['w2'], p['b2'].reshape(1, d),
      p['ln_g'].reshape(1, d), p['ln_b'].reshape(1, d))


def _pack_bf16(y):
    a = lax.bitcast_convert_type(y[:, :64].astype(jnp.bfloat16), jnp.uint16)
    b = lax.bitcast_convert_type(y[:, 64:].astype(jnp.bfloat16), jnp.uint16)
    packed = a.astype(jnp.uint32) | (b.astype(jnp.uint32) << 16)
    return lax.bitcast_convert_type(packed, jnp.int32)


def _unpack_bf16(p):
    u = lax.bitcast_convert_type(p, jnp.uint32)
    lo = lax.bitcast_convert_type(u << 16, F32)
    hi = lax.bitcast_convert_type(u & jnp.uint32(0xFFFF0000), F32)
    return jnp.concatenate([lo, hi], axis=1)


def _pair_linear_body(x_ref, wi_ref, wj_ref, yd_ref, ys_ref):
    x = x_ref[...]
    yd_ref[...] = _dot(x, wi_ref[...])
    ys_ref[...] = _dot(x, wj_ref[...])


def _pair_linear(x, wi, wj):
    n, d = x.shape
    blk = 2000
    return pl.pallas_call(
        _pair_linear_body,
        grid=(n // blk,),
        in_specs=[
            pl.BlockSpec((blk, d), lambda i: (i, 0)),
            pl.BlockSpec((d, d), lambda i: (0, 0)),
            pl.BlockSpec((d, d), lambda i: (0, 0)),
        ],
        out_specs=[
            pl.BlockSpec((blk, d), lambda i: (i, 0)),
            pl.BlockSpec((blk, d), lambda i: (i, 0)),
        ],
        out_shape=[
            jax.ShapeDtypeStruct((n, d), F32),
            jax.ShapeDtypeStruct((n, d), F32),
        ],
    )(x, wi, wj)


def _edge_body(gd_ref, gs_ref, ea_ref, w1e_ref, b1_ref, w2_ref, b2_ref,
               g_ref, be_ref, o_ref):
    ea = ea_ref[...]
    h = gd_ref[...] + gs_ref[...] + b1_ref[...]
    h = h + _dot(ea, w1e_ref[...])
    h = h * jax.nn.sigmoid(h)
    h = _dot(h, w2_ref[...]) + b2_ref[...]
    o_ref[...] = _layernorm(h, g_ref[...], be_ref[...]) + ea


def _edge_mlp(gd, gs, ea, p, rows, g_off, ea_off):
    d = ea.shape[1]
    w1e = p['w1'][2 * d:, :]
    blk = 8000
    goff = g_off // blk
    eoff = ea_off // blk
    return pl.pallas_call(
        _edge_body,
        grid=(rows // blk,),
        in_specs=[
            pl.BlockSpec((blk, d), lambda i: (i + goff, 0)),
            pl.BlockSpec((blk, d), lambda i: (i + goff, 0)),
            pl.BlockSpec((blk, d), lambda i: (i + eoff, 0)),
            pl.BlockSpec((d, d), lambda i: (0, 0)),
            pl.BlockSpec((1, d), lambda i: (0, 0)),
            pl.BlockSpec((d, d), lambda i: (0, 0)),
            pl.BlockSpec((1, d), lambda i: (0, 0)),
            pl.BlockSpec((1, d), lambda i: (0, 0)),
            pl.BlockSpec((1, d), lambda i: (0, 0)),
        ],
        out_specs=pl.BlockSpec((blk, d), lambda i: (i, 0)),
        out_shape=jax.ShapeDtypeStruct((rows, d), F32),
    )(gd, gs, ea, w1e, p['b1'].reshape(1, d), p['w2'], p['b2'].reshape(1, d),
      p['ln_g'].reshape(1, d), p['ln_b'].reshape(1, d))


def _node_body(x_ref, p0_ref, p1_ref, p2_ref, p3_ref, v1x_ref, v1a_ref,
               c1_ref, v2_ref, c2_ref, g_ref, be_ref, o_ref):
    x = x_ref[...]
    agg = (p0_ref[...] + p1_ref[...]) + (p2_ref[...] + p3_ref[...])
    h = _dot(x, v1x_ref[...]) + c1_ref[...]
    h = h + _dot(agg, v1a_ref[...])
    h = h * jax.nn.sigmoid(h)
    h = _dot(h, v2_ref[...]) + c2_ref[...]
    o_ref[...] = _layernorm(h, g_ref[...], be_ref[...]) + x


def _node_mlp(x, parts_list, p):
    n, d = x.shape
    v1x = p['w1'][:d, :]
    v1a = p['w1'][d:, :]
    blk = 2000
    p0, p1 = parts_list[0][0], parts_list[0][1]
    p2, p3 = parts_list[1][0], parts_list[1][1]
    return pl.pallas_call(
        _node_body,
        grid=(n // blk,),
        in_specs=[
            pl.BlockSpec((blk, d), lambda i: (i, 0)),
            pl.BlockSpec((blk, d), lambda i: (i, 0)),
            pl.BlockSpec((blk, d), lambda i: (i, 0)),
            pl.BlockSpec((blk, d), lambda i: (i, 0)),
            pl.BlockSpec((blk, d), lambda i: (i, 0)),
            pl.BlockSpec((d, d), lambda i: (0, 0)),
            pl.BlockSpec((d, d), lambda i: (0, 0)),
            pl.BlockSpec((1, d), lambda i: (0, 0)),
            pl.BlockSpec((d, d), lambda i: (0, 0)),
            pl.BlockSpec((1, d), lambda i: (0, 0)),
            pl.BlockSpec((1, d), lambda i: (0, 0)),
            pl.BlockSpec((1, d), lambda i: (0, 0)),
        ],
        out_specs=pl.BlockSpec((blk, d), lambda i: (i, 0)),
        out_shape=jax.ShapeDtypeStruct((n, d), F32),
    )(x, p0, p1, p2, p3, v1x, v1a, p['b1'].reshape(1, d), p['w2'],
      p['b2'].reshape(1, d), p['ln_g'].reshape(1, d), p['ln_b'].reshape(1, d))


_NC = 2
_NS = 16
_NW = _NC * _NS
_W = 128


def _sc_gather_spmem(table, idx2d):
    n, d = table.shape
    n_edges = idx2d.shape[1]
    nblk = n_edges // _W
    main = (nblk // _NW) * _NW
    tail_blocks = nblk - main
    rows_per_sub = (n // _NS) // 8 * 8
    tail_start = rows_per_sub * _NS
    tail_rows = n - tail_start
    mesh = plsc.VectorSubcoreMesh(core_axis_name="c", subcore_axis_name="s")

    @functools.partial(
        pl.kernel, mesh=mesh,
        out_type=jax.ShapeDtypeStruct((n_edges, d), table.dtype),
        scratch_types=[
            pltpu.VMEM_SHARED((n, d), table.dtype),
            pltpu.VMEM((_W,), jnp.int32),
            pltpu.VMEM((_W, d), table.dtype),
        ],
    )
    def k(tbl_hbm, di_hbm, out_hbm, tbl_sh, idx_tv, rows_tv):
        cid = lax.axis_index("c")
        sid = lax.axis_index("s")
        r0 = sid * rows_per_sub
        pltpu.sync_copy(tbl_hbm.at[pl.ds(r0, rows_per_sub)],
                        tbl_sh.at[pl.ds(r0, rows_per_sub)])

        @pl.when(sid == 0)
        def _():
            pltpu.sync_copy(tbl_hbm.at[pl.ds(tail_start, tail_rows)],
                            tbl_sh.at[pl.ds(tail_start, tail_rows)])

        plsc.subcore_barrier()

        def body(di_v, o_v):
            pltpu.sync_copy(tbl_sh.at[di_v.at[0]], o_v)

        pltpu.emit_pipeline(
            body,
            grid=(main,),
            in_specs=[pl.BlockSpec((1, _W), lambda i: (0, i))],
            out_specs=[pl.BlockSpec((_W, d), lambda i: (i, 0))],
            core_axis_name=("c", "s"),
            dimension_semantics=(pltpu.PARALLEL,),
        )(di_hbm, out_hbm)

        wid = sid * _NC + cid

        @pl.when(wid < tail_blocks)
        def _():
            base = (main + wid) * _W
            pltpu.sync_copy(di_hbm.at[0].at[pl.ds(base, _W)], idx_tv)
            pltpu.sync_copy(tbl_sh.at[idx_tv], rows_tv)
            pltpu.sync_copy(rows_tv, out_hbm.at[pl.ds(base, _W)])

    return k(table, idx2d)


def _sc_segsum(msgs, idx2d, zeros):
    n_edges, d = msgs.shape
    n = zeros.shape[0]
    nblk = n_edges // _W
    main = (nblk // _NW) * _NW
    tail_blocks = nblk - main
    rows_per_sub = (n // _NS) // 8 * 8
    tail_start = rows_per_sub * _NS
    tail = n - tail_start
    mesh = plsc.VectorSubcoreMesh(core_axis_name="c", subcore_axis_name="s")

    @functools.partial(
        pl.kernel, mesh=mesh,
        out_type=jax.ShapeDtypeStruct((_NC, n, d), F32),
        scratch_types=[
            pltpu.VMEM((_W,), jnp.int32),
            pltpu.VMEM((_W, d), F32),
            pltpu.VMEM_SHARED((n, d), F32),
        ],
    )
    def k(msgs_hbm, idx_hbm, zeros_hbm, out_hbm, idx_v, rows_v, agg_sh):
        cid = lax.axis_index("c")
        sid = lax.axis_index("s")
        wid = sid * _NC + cid
        r0 = sid * rows_per_sub
        pltpu.sync_copy(zeros_hbm.at[pl.ds(r0, rows_per_sub)],
                        agg_sh.at[pl.ds(r0, rows_per_sub)])

        @pl.when(sid == 0)
        def _():
            pltpu.sync_copy(zeros_hbm.at[pl.ds(tail_start, tail)],
                            agg_sh.at[pl.ds(tail_start, tail)])

        plsc.subcore_barrier()

        def body(m_v, di_v):
            pltpu.sync_copy(m_v, agg_sh.at[di_v.at[0]], add=True)

        pltpu.emit_pipeline(
            body,
            grid=(main,),
            in_specs=[
                pl.BlockSpec((_W, d), lambda i: (i, 0)),
                pl.BlockSpec((1, _W), lambda i: (0, i)),
            ],
            out_specs=[],
            core_axis_name=("c", "s"),
            dimension_semantics=(pltpu.PARALLEL,),
        )(msgs_hbm, idx_hbm)

        @pl.when(wid < tail_blocks)
        def _():
            base = (main + wid) * _W
            pltpu.sync_copy(idx_hbm.at[0].at[pl.ds(base, _W)], idx_v)
            pltpu.sync_copy(msgs_hbm.at[pl.ds(base, _W)], rows_v)
            pltpu.sync_copy(rows_v, agg_sh.at[idx_v], add=True)

        plsc.subcore_barrier()
        pltpu.sync_copy(agg_sh.at[pl.ds(r0, rows_per_sub)],
                        out_hbm.at[cid].at[pl.ds(r0, rows_per_sub)])

        @pl.when(sid == 0)
        def _():
            pltpu.sync_copy(agg_sh.at[pl.ds(tail_start, tail)],
                            out_hbm.at[cid].at[pl.ds(tail_start, tail)])

    return k(msgs, idx2d, zeros)


def kernel(x, edge_attr, edge_index, shapes, emb_params, block_params):
    del shapes
    n, d = x.shape
    n_edges = edge_index.shape[1]
    half = n_edges // 2
    src2d = edge_index[0].reshape(1, -1)
    dst2d = edge_index[1].reshape(1, -1)
    srcs = [src2d[:, :half], src2d[:, half:]]
    dsts = [dst2d[:, :half], dst2d[:, half:]]
    zeros = jnp.zeros((n, d), F32)

    h1 = jnp.dot(edge_attr.astype(jnp.bfloat16),
                 emb_params['w1'].astype(jnp.bfloat16),
                 preferred_element_type=F32) + emb_params['b1']
    ea_full = _emb_mlp(h1, emb_params, 0, n_edges)
    eas, ea_offs = [ea_full, ea_full], [0, half]
    x_out = x
    for li, p in enumerate(block_params):
        w1 = p['edge_mlp']['w1']
        yd, ys = _pair_linear(x_out, w1[:d, :], w1[d:2 * d, :])
        if li == 0:
            gd = _sc_gather_spmem(yd, dst2d)
            gs = _sc_gather_spmem(ys, src2d)
            gpair = [(gd, gs, half), (gd, gs, half)]
            g_offs = [0, half]
        else:
            gpair = [( _sc_gather_spmem(yd, dsts[k]),
                       _sc_gather_spmem(ys, srcs[k]), half) for k in range(2)]
            g_offs = [0, 0]
        ens, parts = [], []
        for k in range(2):
            gdk, gsk, rows = gpair[k]
            ens.append(_edge_mlp(gdk, gsk, eas[k], p['edge_mlp'], rows,
                                 g_offs[k], ea_offs[k]))
            parts.append(_sc_segsum(ens[k], dsts[k], zeros))
        x_out = _node_mlp(x_out, parts, p['node_mlp'])
        eas, ea_offs = ens, [0, 0]
    return (x_out, jnp.concatenate(eas, axis=0))
